# baseline (device time: 84649 ns/iter reference)
import jax
import jax.numpy as jnp
from jax import lax
from jax.experimental import pallas as pl
from jax.experimental.pallas import tpu as pltpu

N_DEV = 4


def kernel(A, B):
    m_per, k = A.shape
    _, n = B.shape
    half = m_per // 2

    def body(a_ref, b_ref, out_ref, ag_ref, bb_ref, stage_ref,
             send_sems, recv_sems, out_sems):
        my = lax.axis_index("i")
        left = lax.rem(my + N_DEV - 1, N_DEV)
        right = lax.rem(my + 1, N_DEV)

        def flow(src, dst, sem, dev):
            return pltpu.make_async_remote_copy(
                src_ref=src, dst_ref=dst,
                send_sem=send_sems.at[sem], recv_sem=recv_sems.at[sem],
                device_id=(dev,), device_id_type=pl.DeviceIdType.MESH,
            )

        top = pl.ds(0, half)
        bot = pl.ds(half, half)

        ag_ref[3] = a_ref[...].astype(jnp.bfloat16)
        bb_ref[...] = b_ref[...].astype(jnp.bfloat16)

        pending = [None, None]

        def emit(piece, src_rows_ref, origin, row_off):
            slot = piece % 2
            if pending[slot] is not None:
                pending[slot].wait()
            stage_ref[slot] = jnp.dot(
                src_rows_ref, bb_ref[...], preferred_element_type=jnp.float32
            )
            o = lax.rem(origin + 2 * N_DEV, N_DEV)
            cp = pltpu.make_async_copy(
                stage_ref.at[slot],
                out_ref.at[pl.ds(o * m_per + row_off, half)],
                out_sems.at[slot],
            )
            cp.start()
            pending[slot] = cp

        emit(0, ag_ref[3, top], my, 0)
        emit(1, ag_ref[3, bot], my, half)

        barrier_sem = pltpu.get_barrier_semaphore()
        for nbr in (left, right):
            pl.semaphore_signal(
                barrier_sem, inc=1,
                device_id=(nbr,), device_id_type=pl.DeviceIdType.MESH,
            )
        pl.semaphore_wait(barrier_sem, 2)

        rs1 = flow(ag_ref.at[3, top], ag_ref.at[0, top], 0, right)
        rs1.start()
        ls1 = flow(ag_ref.at[3, bot], ag_ref.at[1, bot], 1, left)
        ls1.start()

        rs1.wait_recv()
        rf = flow(ag_ref.at[0, top], ag_ref.at[2, top], 2, right)
        rf.start()
        rs2 = flow(ag_ref.at[3, bot], ag_ref.at[0, bot], 4, right)
        rs2.start()
        ls1.wait_recv()
        lf = flow(ag_ref.at[1, bot], ag_ref.at[2, bot], 3, left)
        lf.start()
        ls2 = flow(ag_ref.at[3, top], ag_ref.at[1, top], 5, left)
        ls2.start()

        emit(2, ag_ref[0, top], my - 1, 0)
        emit(3, ag_ref[1, bot], my + 1, half)
        rf.wait_recv()
        emit(4, ag_ref[2, top], my + 2, 0)
        lf.wait_recv()
        emit(5, ag_ref[2, bot], my + 2, half)
        rs2.wait_recv()
        emit(6, ag_ref[0, bot], my - 1, half)
        ls2.wait_recv()
        emit(7, ag_ref[1, top], my + 1, 0)

        for f in (rs1, ls1, rf, lf, rs2, ls2):
            f.wait_send()
        for cp in pending:
            cp.wait()

    return pl.pallas_call(
        body,
        out_shape=jax.ShapeDtypeStruct((N_DEV * m_per, n), jnp.float32),
        in_specs=[
            pl.BlockSpec(memory_space=pltpu.VMEM),
            pl.BlockSpec(memory_space=pltpu.VMEM),
        ],
        out_specs=pl.BlockSpec(memory_space=pl.ANY),
        scratch_shapes=[
            pltpu.VMEM((N_DEV, m_per, k), jnp.bfloat16),
            pltpu.VMEM((k, n), jnp.bfloat16),
            pltpu.VMEM((2, half, n), jnp.float32),
            pltpu.SemaphoreType.DMA((6,)),
            pltpu.SemaphoreType.DMA((6,)),
            pltpu.SemaphoreType.DMA((2,)),
        ],
        compiler_params=pltpu.CompilerParams(
            collective_id=0,
            vmem_limit_bytes=100 * 1024 * 1024,
        ),
    )(A, B)


# device time: 80216 ns/iter; 1.0553x vs baseline; 1.0553x over previous
import jax
import jax.numpy as jnp
from jax import lax
from jax.experimental import pallas as pl
from jax.experimental.pallas import tpu as pltpu

N_DEV = 4


def kernel(A, B):
    m_per, k = A.shape
    _, n = B.shape
    half = m_per // 2

    def body(a_ref, b_ref, out_ref, ag_ref, bb_ref, stage_ref,
             send_sems, recv_sems, out_sems):
        my = lax.axis_index("i")
        left = lax.rem(my + N_DEV - 1, N_DEV)
        right = lax.rem(my + 1, N_DEV)

        def flow(src, dst, sem, dev):
            return pltpu.make_async_remote_copy(
                src_ref=src, dst_ref=dst,
                send_sem=send_sems.at[sem], recv_sem=recv_sems.at[sem],
                device_id=(dev,), device_id_type=pl.DeviceIdType.MESH,
            )

        top = pl.ds(0, half)
        bot = pl.ds(half, half)

        ag_ref[3] = a_ref[...].astype(jnp.bfloat16)

        barrier_sem = pltpu.get_barrier_semaphore()
        for nbr in (left, right):
            pl.semaphore_signal(
                barrier_sem, inc=1,
                device_id=(nbr,), device_id_type=pl.DeviceIdType.MESH,
            )
        pl.semaphore_wait(barrier_sem, 2)

        rs1 = flow(ag_ref.at[3, top], ag_ref.at[0, top], 0, right)
        rs1.start()
        ls1 = flow(ag_ref.at[3, bot], ag_ref.at[1, bot], 1, left)
        ls1.start()

        bb_ref[...] = b_ref[...].astype(jnp.bfloat16)

        pending = [None, None]

        def emit(piece, src_rows_ref, origin, row_off):
            slot = piece % 2
            if pending[slot] is not None:
                pending[slot].wait()
            stage_ref[slot] = jnp.dot(
                src_rows_ref, bb_ref[...], preferred_element_type=jnp.float32
            )
            o = lax.rem(origin + 2 * N_DEV, N_DEV)
            cp = pltpu.make_async_copy(
                stage_ref.at[slot],
                out_ref.at[pl.ds(o * m_per + row_off, half)],
                out_sems.at[slot],
            )
            cp.start()
            pending[slot] = cp

        emit(0, ag_ref[3, top], my, 0)
        emit(1, ag_ref[3, bot], my, half)

        rs1.wait_recv()
        rf = flow(ag_ref.at[0, top], ag_ref.at[2, top], 2, right)
        rf.start()
        rs2 = flow(ag_ref.at[3, bot], ag_ref.at[0, bot], 4, right)
        rs2.start()
        ls1.wait_recv()
        lf = flow(ag_ref.at[1, bot], ag_ref.at[2, bot], 3, left)
        lf.start()
        ls2 = flow(ag_ref.at[3, top], ag_ref.at[1, top], 5, left)
        ls2.start()

        emit(2, ag_ref[0, top], my - 1, 0)
        emit(3, ag_ref[1, bot], my + 1, half)
        rf.wait_recv()
        emit(4, ag_ref[2, top], my + 2, 0)
        lf.wait_recv()
        emit(5, ag_ref[2, bot], my + 2, half)
        rs2.wait_recv()
        emit(6, ag_ref[0, bot], my - 1, half)
        ls2.wait_recv()
        emit(7, ag_ref[1, top], my + 1, 0)

        for f in (rs1, ls1, rf, lf, rs2, ls2):
            f.wait_send()
        for cp in pending:
            cp.wait()

    return pl.pallas_call(
        body,
        out_shape=jax.ShapeDtypeStruct((N_DEV * m_per, n), jnp.float32),
        in_specs=[
            pl.BlockSpec(memory_space=pltpu.VMEM),
            pl.BlockSpec(memory_space=pltpu.VMEM),
        ],
        out_specs=pl.BlockSpec(memory_space=pl.ANY),
        scratch_shapes=[
            pltpu.VMEM((N_DEV, m_per, k), jnp.bfloat16),
            pltpu.VMEM((k, n), jnp.bfloat16),
            pltpu.VMEM((2, half, n), jnp.float32),
            pltpu.SemaphoreType.DMA((6,)),
            pltpu.SemaphoreType.DMA((6,)),
            pltpu.SemaphoreType.DMA((2,)),
        ],
        compiler_params=pltpu.CompilerParams(
            collective_id=0,
            vmem_limit_bytes=100 * 1024 * 1024,
        ),
    )(A, B)
